# Initial kernel scaffold; baseline (speedup 1.0000x reference)
import jax
import jax.numpy as jnp
from jax.experimental import pallas as pl


def kernel(x, edge_index, W1, b1, W2, b2, W3, b3, Wq, bq, Wk, bk, Wv, bv, Wo, bo, Wf1, bf1, Wf2, bf2, Wp1, bp1, Wp2, bp2):
    raise NotImplementedError("write your pallas kernel here")


# speedup vs baseline: 9.0472x; 9.0472x over previous
import functools

import jax
import jax.numpy as jnp
from jax.experimental import pallas as pl

N = 4096
DIN = 128
DH = 256
HEADS = 8
DHEAD = 32
BLK = 512
NBLK = N // BLK

_INTERPRET = False


def _u1_body(x_ref, w_ref, deg_ref, u_ref):
    z = jnp.dot(x_ref[...], w_ref[...], preferred_element_type=jnp.float32)
    dinv = jax.lax.rsqrt(deg_ref[...])
    u_ref[...] = dinv * z


def _u1(x, W1, deg):
    return pl.pallas_call(
        _u1_body,
        grid=(NBLK,),
        in_specs=[
            pl.BlockSpec((BLK, DIN), lambda r: (r, 0)),
            pl.BlockSpec((DIN, DH), lambda r: (0, 0)),
            pl.BlockSpec((BLK, 1), lambda r: (r, 0)),
        ],
        out_specs=pl.BlockSpec((BLK, DH), lambda r: (r, 0)),
        out_shape=jax.ShapeDtypeStruct((N, DH), jnp.float32),
        interpret=_INTERPRET,
    )(x, W1, deg)


def _gcn_body(a_ref, uf_ref, ub_ref, deg_ref, b_ref, wn_ref, un_ref, *, relu):
    agg = jnp.dot(a_ref[...], uf_ref[...], preferred_element_type=jnp.float32)
    agg = agg + ub_ref[...]
    dinv = jax.lax.rsqrt(deg_ref[...])
    h = dinv * agg + b_ref[...]
    if relu:
        h = jnp.maximum(h, 0.0)
    z = jnp.dot(h, wn_ref[...], preferred_element_type=jnp.float32)
    un_ref[...] = dinv * z


def _gcn_mid(A, U, deg, b, Wn, relu):
    return pl.pallas_call(
        functools.partial(_gcn_body, relu=relu),
        grid=(NBLK,),
        in_specs=[
            pl.BlockSpec((BLK, N), lambda r: (r, 0)),
            pl.BlockSpec((N, DH), lambda r: (0, 0)),
            pl.BlockSpec((BLK, DH), lambda r: (r, 0)),
            pl.BlockSpec((BLK, 1), lambda r: (r, 0)),
            pl.BlockSpec((1, DH), lambda r: (0, 0)),
            pl.BlockSpec((DH, DH), lambda r: (0, 0)),
        ],
        out_specs=pl.BlockSpec((BLK, DH), lambda r: (r, 0)),
        out_shape=jax.ShapeDtypeStruct((N, DH), jnp.float32),
        interpret=_INTERPRET,
    )(A, U, U, deg, b, Wn)


def _gcn3_body(a_ref, uf_ref, ub_ref, deg_ref, b_ref,
               wq_ref, bq_ref, wk_ref, bk_ref, wv_ref, bv_ref,
               h_ref, q_ref, k_ref, v_ref):
    agg = jnp.dot(a_ref[...], uf_ref[...], preferred_element_type=jnp.float32)
    agg = agg + ub_ref[...]
    dinv = jax.lax.rsqrt(deg_ref[...])
    h = dinv * agg + b_ref[...]
    h_ref[...] = h
    q_ref[...] = jnp.dot(h, wq_ref[...], preferred_element_type=jnp.float32) + bq_ref[...]
    k_ref[...] = jnp.dot(h, wk_ref[...], preferred_element_type=jnp.float32) + bk_ref[...]
    v_ref[...] = jnp.dot(h, wv_ref[...], preferred_element_type=jnp.float32) + bv_ref[...]


def _gcn_last(A, U, deg, b, Wq, bq, Wk, bk, Wv, bv):
    full = pl.BlockSpec((DH, DH), lambda r: (0, 0))
    bias = pl.BlockSpec((1, DH), lambda r: (0, 0))
    row = pl.BlockSpec((BLK, DH), lambda r: (r, 0))
    return pl.pallas_call(
        _gcn3_body,
        grid=(NBLK,),
        in_specs=[
            pl.BlockSpec((BLK, N), lambda r: (r, 0)),
            pl.BlockSpec((N, DH), lambda r: (0, 0)),
            row,
            pl.BlockSpec((BLK, 1), lambda r: (r, 0)),
            bias, full, bias, full, bias, full, bias,
        ],
        out_specs=[row, row, row, row],
        out_shape=[jax.ShapeDtypeStruct((N, DH), jnp.float32)] * 4,
        interpret=_INTERPRET,
    )(A, U, U, deg, b, Wq, bq, Wk, bk, Wv, bv)


def _mha_body(q_ref, k_ref, v_ref, o_ref):
    q = q_ref[0]
    s = jax.lax.dot_general(q, k_ref[0], (((1,), (1,)), ((), ())),
                            preferred_element_type=jnp.float32)
    s = s * (1.0 / (DHEAD ** 0.5))
    m = jnp.max(s, axis=-1, keepdims=True)
    p = jnp.exp(s - m)
    denom = jnp.sum(p, axis=-1, keepdims=True)
    o = jnp.dot(p, v_ref[0], preferred_element_type=jnp.float32)
    o_ref[0] = o / denom


def _mha(Q, K, V):
    return pl.pallas_call(
        _mha_body,
        grid=(HEADS, NBLK),
        in_specs=[
            pl.BlockSpec((1, BLK, DHEAD), lambda h, r: (h, r, 0)),
            pl.BlockSpec((1, N, DHEAD), lambda h, r: (h, 0, 0)),
            pl.BlockSpec((1, N, DHEAD), lambda h, r: (h, 0, 0)),
        ],
        out_specs=pl.BlockSpec((1, BLK, DHEAD), lambda h, r: (h, r, 0)),
        out_shape=jax.ShapeDtypeStruct((HEADS, N, DHEAD), jnp.float32),
        interpret=_INTERPRET,
    )(Q, K, V)


def _post_body(h_ref, o_ref, wo_ref, bo_ref, wf1_ref, bf1_ref, wf2_ref, bf2_ref,
               wp1_ref, bp1_ref, wp2_ref, bp2_ref, risk_ref, pat_ref):
    attn = jnp.dot(o_ref[...], wo_ref[...], preferred_element_type=jnp.float32) + bo_ref[...]
    h = h_ref[...] + attn
    t = jnp.maximum(jnp.dot(h, wf1_ref[...], preferred_element_type=jnp.float32) + bf1_ref[...], 0.0)
    rl = jnp.dot(t, wf2_ref[...], preferred_element_type=jnp.float32) + bf2_ref[...]
    m = jnp.max(rl, axis=-1, keepdims=True)
    e = jnp.exp(rl - m)
    risk_ref[...] = e / jnp.sum(e, axis=-1, keepdims=True)
    t2 = jnp.maximum(jnp.dot(h, wp1_ref[...], preferred_element_type=jnp.float32) + bp1_ref[...], 0.0)
    pl_ = jnp.dot(t2, wp2_ref[...], preferred_element_type=jnp.float32) + bp2_ref[...]
    pat_ref[...] = 1.0 / (1.0 + jnp.exp(-pl_))


def _post(h3, O, Wo, bo, Wf1, bf1, Wf2, bf2, Wp1, bp1, Wp2, bp2):
    row = pl.BlockSpec((BLK, DH), lambda r: (r, 0))
    const = lambda shape: pl.BlockSpec(shape, lambda r: (0, 0))
    return pl.pallas_call(
        _post_body,
        grid=(NBLK,),
        in_specs=[
            row, row,
            const((DH, DH)), const((1, DH)),
            const((DH, DH // 2)), const((1, DH // 2)),
            const((DH // 2, 3)), const((1, 3)),
            const((DH, DH // 2)), const((1, DH // 2)),
            const((DH // 2, 10)), const((1, 10)),
        ],
        out_specs=[
            pl.BlockSpec((BLK, 3), lambda r: (r, 0)),
            pl.BlockSpec((BLK, 10), lambda r: (r, 0)),
        ],
        out_shape=[
            jax.ShapeDtypeStruct((N, 3), jnp.float32),
            jax.ShapeDtypeStruct((N, 10), jnp.float32),
        ],
        interpret=_INTERPRET,
    )(h3, O, Wo, bo, Wf1, bf1, Wf2, bf2, Wp1, bp1, Wp2, bp2)


def _build_adjacency(edge_index):
    src = edge_index[0]
    dst = edge_index[1]
    A = jnp.zeros((N, N), jnp.float32).at[dst, src].add(1.0)
    deg = jnp.zeros((N,), jnp.float32).at[dst].add(1.0) + 1.0
    return A, deg


def kernel(x, edge_index, W1, b1, W2, b2, W3, b3, Wq, bq, Wk, bk, Wv, bv, Wo, bo,
           Wf1, bf1, Wf2, bf2, Wp1, bp1, Wp2, bp2):
    A, deg = _build_adjacency(edge_index)
    deg = deg.reshape(N, 1)
    r2 = lambda b: b.reshape(1, -1)

    U1 = _u1(x, W1, deg)
    U2 = _gcn_mid(A, U1, deg, r2(b1), W2, relu=True)
    U3 = _gcn_mid(A, U2, deg, r2(b2), W3, relu=True)
    h3, Q, K, V = _gcn_last(A, U3, deg, r2(b3), Wq, r2(bq), Wk, r2(bk), Wv, r2(bv))
    hm = lambda M: M.reshape(N, HEADS, DHEAD).transpose(1, 0, 2)
    O = _mha(hm(Q), hm(K), hm(V))
    O = O.transpose(1, 0, 2).reshape(N, DH)
    risk, pattern = _post(h3, O, Wo, r2(bo), Wf1, r2(bf1), Wf2, r2(bf2),
                          Wp1, r2(bp1), Wp2, r2(bp2))
    return (risk, pattern)
